# baseline (speedup 1.0000x reference)
import jax
import jax.numpy as jnp
import numpy as np
from jax.experimental import pallas as pl
from jax.experimental.pallas import tpu as pltpu

_MAX_FRAMES = 12
_NUM_SAMPLES = 256
_SIGMA = 0.05

_CONST_CACHE = {}


def _selection_constants(BF, L, k):
    cache_key = (BF, L, k)
    if cache_key in _CONST_CACHE:
        return _CONST_CACHE[cache_key]
    with jax.ensure_compile_time_eval():
        _CONST_CACHE[cache_key] = _build_selection_constants(BF, L, k)
    return _CONST_CACHE[cache_key]


def _build_selection_constants(BF, L, k):
    ns = _NUM_SAMPLES
    nk = L - k
    noise = jax.random.normal(jax.random.key(42), (BF, ns, L), dtype=jnp.float32)
    scores = jnp.ones((BF, L), jnp.float32)
    perturbed = scores[:, None, :] + noise * _SIGMA
    _, idx = jax.lax.top_k(perturbed, k)
    sidx_j = jnp.sort(idx, axis=-1)
    sidx = np.asarray(sidx_j)
    b_i = np.arange(BF)[:, None, None]
    k_i = np.arange(k)[None, None, :]
    flat = ((b_i * k + k_i) * L + sidx).ravel()
    cnt = np.bincount(flat, minlength=BF * k * L).reshape(BF, k, L)
    a1 = cnt.astype(np.float32) / ns

    mask = jnp.ones((BF, ns, L), jnp.float32)
    s_i = jnp.arange(ns)[None, :, None]
    mask = mask.at[jnp.arange(BF)[:, None, None], s_i, sidx_j].set(0.0)
    nt_idx = np.asarray(jnp.argsort(1.0 - mask, axis=-1)[..., :nk])
    j_i = np.arange(nk)[None, None, :]
    flat2 = ((b_i * nk + j_i) * L + nt_idx).ravel()
    cnt2 = np.bincount(flat2, minlength=BF * nk * L).reshape(BF, nk, L)
    a2 = cnt2.astype(np.float32) / ns

    n = int(np.random.default_rng(0).integers(1, nk))
    rv = jax.random.uniform(jax.random.key(7), (BF, nk))
    _, midx = jax.lax.top_k(-rv, n)
    m = np.zeros((BF, nk), np.float32)
    m[np.arange(BF)[:, None], np.asarray(midx)] = 1.0
    a3 = a2 * m[:, :, None]
    a1_bf = np.asarray(jnp.asarray(a1, dtype=jnp.bfloat16))
    a2_bf = np.asarray(jnp.asarray(a2, dtype=jnp.bfloat16))
    a3_bf = np.asarray(jnp.asarray(a3, dtype=jnp.bfloat16))
    return (a1_bf, a2_bf, a3_bf)


_BATCH_BLOCK = 16


def _select_body(a1_ref, a2_ref, a3_ref, x_ref, sel_ref, nsel_ref, mt_ref):
    for i in range(_BATCH_BLOCK):
        x = x_ref[i]
        sel_ref[i] = jnp.dot(a1_ref[i].astype(jnp.float32), x,
                             preferred_element_type=jnp.float32)
        nsel_ref[i] = jnp.dot(a2_ref[i].astype(jnp.float32), x,
                              preferred_element_type=jnp.float32)
        mt_ref[i] = jnp.dot(a3_ref[i].astype(jnp.float32), x,
                            preferred_element_type=jnp.float32)


def kernel(x, ln_g, ln_b, W1, W2, W3):
    BF, L, D = x.shape
    k = L // 4
    nk = L - k
    a1_np, a2_np, a3_np = _selection_constants(BF, L, k)
    a1 = jnp.asarray(a1_np)
    a2 = jnp.asarray(a2_np)
    a3 = jnp.asarray(a3_np)
    BB = _BATCH_BLOCK
    sel, nsel, mt = pl.pallas_call(
        _select_body,
        grid=(BF // BB,),
        in_specs=[
            pl.BlockSpec((BB, k, L), lambda b: (b, 0, 0)),
            pl.BlockSpec((BB, nk, L), lambda b: (b, 0, 0)),
            pl.BlockSpec((BB, nk, L), lambda b: (b, 0, 0)),
            pl.BlockSpec((BB, L, D), lambda b: (b, 0, 0)),
        ],
        out_specs=[
            pl.BlockSpec((BB, k, D), lambda b: (b, 0, 0)),
            pl.BlockSpec((BB, nk, D), lambda b: (b, 0, 0)),
            pl.BlockSpec((BB, nk, D), lambda b: (b, 0, 0)),
        ],
        out_shape=[
            jax.ShapeDtypeStruct((BF, k, D), x.dtype),
            jax.ShapeDtypeStruct((BF, nk, D), x.dtype),
            jax.ShapeDtypeStruct((BF, nk, D), x.dtype),
        ],
        compiler_params=pltpu.CompilerParams(
            dimension_semantics=("parallel",),
        ),
    )(a1, a2, a3, x)
    return sel, nsel, mt


# speedup vs baseline: 1.0094x; 1.0094x over previous
import jax
import jax.numpy as jnp
import numpy as np
from jax.experimental import pallas as pl
from jax.experimental.pallas import tpu as pltpu

_MAX_FRAMES = 12
_NUM_SAMPLES = 256
_SIGMA = 0.05

_CONST_CACHE = {}


def _selection_constants(BF, L, k):
    cache_key = (BF, L, k)
    if cache_key in _CONST_CACHE:
        return _CONST_CACHE[cache_key]
    with jax.ensure_compile_time_eval():
        _CONST_CACHE[cache_key] = _build_selection_constants(BF, L, k)
    return _CONST_CACHE[cache_key]


def _build_selection_constants(BF, L, k):
    ns = _NUM_SAMPLES
    nk = L - k
    noise = jax.random.normal(jax.random.key(42), (BF, ns, L), dtype=jnp.float32)
    scores = jnp.ones((BF, L), jnp.float32)
    perturbed = scores[:, None, :] + noise * _SIGMA
    _, idx = jax.lax.top_k(perturbed, k)
    sidx_j = jnp.sort(idx, axis=-1)
    sidx = np.asarray(sidx_j)
    b_i = np.arange(BF)[:, None, None]
    k_i = np.arange(k)[None, None, :]
    flat = ((b_i * k + k_i) * L + sidx).ravel()
    cnt = np.bincount(flat, minlength=BF * k * L).reshape(BF, k, L)
    a1 = cnt.astype(np.float32) / ns

    mask = jnp.ones((BF, ns, L), jnp.float32)
    s_i = jnp.arange(ns)[None, :, None]
    mask = mask.at[jnp.arange(BF)[:, None, None], s_i, sidx_j].set(0.0)
    nt_idx = np.asarray(jnp.argsort(1.0 - mask, axis=-1)[..., :nk])
    j_i = np.arange(nk)[None, None, :]
    flat2 = ((b_i * nk + j_i) * L + nt_idx).ravel()
    cnt2 = np.bincount(flat2, minlength=BF * nk * L).reshape(BF, nk, L)
    a2 = cnt2.astype(np.float32) / ns

    n = int(np.random.default_rng(0).integers(1, nk))
    rv = jax.random.uniform(jax.random.key(7), (BF, nk))
    _, midx = jax.lax.top_k(-rv, n)
    m = np.zeros((BF, nk), np.float32)
    m[np.arange(BF)[:, None], np.asarray(midx)] = 1.0
    a1_bf = np.asarray(jnp.asarray(a1, dtype=jnp.bfloat16))
    a2_bf = np.asarray(jnp.asarray(a2, dtype=jnp.bfloat16))
    return (a1_bf, a2_bf, m.reshape(BF, nk, 1))


_BATCH_BLOCK = 16


def _select_body(a1_ref, a2_ref, m_ref, x_ref, sel_ref, nsel_ref, mt_ref):
    for i in range(_BATCH_BLOCK):
        x = x_ref[i]
        sel_ref[i] = jnp.dot(a1_ref[i].astype(jnp.float32), x,
                             preferred_element_type=jnp.float32)
        nsel = jnp.dot(a2_ref[i].astype(jnp.float32), x,
                       preferred_element_type=jnp.float32)
        nsel_ref[i] = nsel
        mt_ref[i] = nsel * m_ref[i]


def kernel(x, ln_g, ln_b, W1, W2, W3):
    BF, L, D = x.shape
    k = L // 4
    nk = L - k
    a1_np, a2_np, m_np = _selection_constants(BF, L, k)
    a1 = jnp.asarray(a1_np)
    a2 = jnp.asarray(a2_np)
    m = jnp.asarray(m_np)
    BB = _BATCH_BLOCK
    sel, nsel, mt = pl.pallas_call(
        _select_body,
        grid=(BF // BB,),
        in_specs=[
            pl.BlockSpec((BB, k, L), lambda b: (b, 0, 0)),
            pl.BlockSpec((BB, nk, L), lambda b: (b, 0, 0)),
            pl.BlockSpec((BB, nk, 1), lambda b: (b, 0, 0)),
            pl.BlockSpec((BB, L, D), lambda b: (b, 0, 0)),
        ],
        out_specs=[
            pl.BlockSpec((BB, k, D), lambda b: (b, 0, 0)),
            pl.BlockSpec((BB, nk, D), lambda b: (b, 0, 0)),
            pl.BlockSpec((BB, nk, D), lambda b: (b, 0, 0)),
        ],
        out_shape=[
            jax.ShapeDtypeStruct((BF, k, D), x.dtype),
            jax.ShapeDtypeStruct((BF, nk, D), x.dtype),
            jax.ShapeDtypeStruct((BF, nk, D), x.dtype),
        ],
        compiler_params=pltpu.CompilerParams(
            dimension_semantics=("parallel",),
        ),
    )(a1, a2, m, x)
    return sel, nsel, mt


# speedup vs baseline: 1.0107x; 1.0013x over previous
import jax
import jax.numpy as jnp
import numpy as np
from jax.experimental import pallas as pl
from jax.experimental.pallas import tpu as pltpu

_NUM_SAMPLES = 256
_SIGMA = 0.05

_CONST_CACHE = {}


def _selection_constants(BF, L, k):
    cache_key = (BF, L, k)
    if cache_key in _CONST_CACHE:
        return _CONST_CACHE[cache_key]
    with jax.ensure_compile_time_eval():
        _CONST_CACHE[cache_key] = _build_selection_constants(BF, L, k)
    return _CONST_CACHE[cache_key]


def _build_selection_constants(BF, L, k):
    ns = _NUM_SAMPLES
    nk = L - k
    noise = jax.random.normal(jax.random.key(42), (BF, ns, L), dtype=jnp.float32)
    scores = jnp.ones((BF, L), jnp.float32)
    perturbed = scores[:, None, :] + noise * _SIGMA
    _, idx = jax.lax.top_k(perturbed, k)
    sidx_j = jnp.sort(idx, axis=-1)
    sidx = np.asarray(sidx_j)
    b_i = np.arange(BF)[:, None, None]
    k_i = np.arange(k)[None, None, :]
    flat = ((b_i * k + k_i) * L + sidx).ravel()
    cnt = np.bincount(flat, minlength=BF * k * L).reshape(BF, k, L)
    a1 = cnt.astype(np.float32) / ns

    mask = jnp.ones((BF, ns, L), jnp.float32)
    s_i = jnp.arange(ns)[None, :, None]
    mask = mask.at[jnp.arange(BF)[:, None, None], s_i, sidx_j].set(0.0)
    nt_idx = np.asarray(jnp.argsort(1.0 - mask, axis=-1)[..., :nk])
    j_i = np.arange(nk)[None, None, :]
    flat2 = ((b_i * nk + j_i) * L + nt_idx).ravel()
    cnt2 = np.bincount(flat2, minlength=BF * nk * L).reshape(BF, nk, L)
    a2 = cnt2.astype(np.float32) / ns

    n = int(np.random.default_rng(0).integers(1, nk))
    rv = jax.random.uniform(jax.random.key(7), (BF, nk))
    _, midx = jax.lax.top_k(-rv, n)
    m = np.zeros((BF, nk), np.float32)
    m[np.arange(BF)[:, None], np.asarray(midx)] = 1.0
    a1_bf = np.asarray(jnp.asarray(a1, dtype=jnp.bfloat16))
    a2_bf = np.asarray(jnp.asarray(a2, dtype=jnp.bfloat16))
    return (a1_bf, a2_bf, m.reshape(BF, nk, 1))


_BATCH_BLOCK = 16


def _select_body(a1_ref, a2_ref, m_ref, x_ref, sel_ref, nsel_ref, mt_ref):
    for i in range(_BATCH_BLOCK):
        x = x_ref[i]
        sel_ref[i] = jnp.dot(a1_ref[i].astype(jnp.float32), x,
                             preferred_element_type=jnp.float32)
        nsel = jnp.dot(a2_ref[i].astype(jnp.float32), x,
                       preferred_element_type=jnp.float32)
        nsel_ref[i] = nsel
        mt_ref[i] = nsel * m_ref[i]


def kernel(x, ln_g, ln_b, W1, W2, W3):
    BF, L, D = x.shape
    k = L // 4
    nk = L - k
    a1_np, a2_np, m_np = _selection_constants(BF, L, k)
    a1 = jnp.asarray(a1_np)
    a2 = jnp.asarray(a2_np)
    m = jnp.asarray(m_np)
    BB = _BATCH_BLOCK
    sel, nsel, mt = pl.pallas_call(
        _select_body,
        grid=(BF // BB,),
        in_specs=[
            pl.BlockSpec((BB, k, L), lambda b: (b, 0, 0)),
            pl.BlockSpec((BB, nk, L), lambda b: (b, 0, 0)),
            pl.BlockSpec((BB, nk, 1), lambda b: (b, 0, 0)),
            pl.BlockSpec((BB, L, D), lambda b: (b, 0, 0)),
        ],
        out_specs=[
            pl.BlockSpec((BB, k, D), lambda b: (b, 0, 0)),
            pl.BlockSpec((BB, nk, D), lambda b: (b, 0, 0)),
            pl.BlockSpec((BB, nk, D), lambda b: (b, 0, 0)),
        ],
        out_shape=[
            jax.ShapeDtypeStruct((BF, k, D), x.dtype),
            jax.ShapeDtypeStruct((BF, nk, D), x.dtype),
            jax.ShapeDtypeStruct((BF, nk, D), x.dtype),
        ],
        compiler_params=pltpu.CompilerParams(
            dimension_semantics=("parallel",),
        ),
    )(a1, a2, m, x)
    return sel, nsel, mt
